# baseline (speedup 1.0000x reference)
import functools

import jax
import jax.numpy as jnp
from jax import lax
from jax.experimental import pallas as pl
from jax.experimental.pallas import tpu as pltpu
from jax.experimental.pallas import tpu_sc as plsc

NC = 2
NS = 16
LANES = 16
G = 64


def _mesh():
    return plsc.VectorSubcoreMesh(core_axis_name="c", subcore_axis_name="s",
                                  num_cores=NC, num_subcores=NS)


MACRO = 3
HHALF = 32


def _make_edge_kernel(n_pad, e_rows, h):
    rows_stage = n_pad // NS
    erows_tile = e_rows // NS
    n_pair = erows_tile // (2 * MACRO)

    @functools.partial(
        pl.kernel,
        out_type=jax.ShapeDtypeStruct((n_pad, h), jnp.float32),
        mesh=_mesh(),
        scratch_types=[
            pltpu.VMEM_SHARED((n_pad, HHALF), jnp.float32),
            pltpu.VMEM((2 * MACRO, 128), jnp.int32),
            pltpu.VMEM((2 * MACRO, 128), jnp.int32),
            pltpu.VMEM((2 * MACRO, 128, HHALF), jnp.float32),
            pltpu.SemaphoreType.DMA,
            pltpu.SemaphoreType.DMA,
            pltpu.SemaphoreType.DMA,
            pltpu.SemaphoreType.DMA,
        ],
        compiler_params=pltpu.CompilerParams(use_tc_tiling_on_sc=False),
    )
    def edge_kernel(q0_hbm, q1_hbm, src_hbm, dst_hbm, zeros_hbm, s_hbm,
                    acc_sp, sbuf, dbuf, msg, gsa, gsb, ssa, ssb):
        c = lax.axis_index("c")
        s = lax.axis_index("s")
        r0 = s * rows_stage
        e0 = s * erows_tile

        def run_half(q_hbm, f0):
            pltpu.sync_copy(zeros_hbm.at[pl.ds(r0, rows_stage), :],
                            acc_sp.at[pl.ds(r0, rows_stage), :])
            plsc.subcore_barrier()

            def pair(pi, carry):
                row0 = e0 + pi * (2 * MACRO)
                pltpu.sync_copy(src_hbm.at[pl.ds(row0, 2 * MACRO), :], sbuf)
                pltpu.sync_copy(dst_hbm.at[pl.ds(row0, 2 * MACRO), :], dbuf)
                ga = [pltpu.async_copy(q_hbm.at[sbuf.at[j]], msg.at[j], gsa)
                      for j in range(MACRO)]
                gb = [pltpu.async_copy(q_hbm.at[sbuf.at[j]], msg.at[j], gsb)
                      for j in range(MACRO, 2 * MACRO)]
                for cp in ga:
                    cp.wait()
                sa = [pltpu.async_copy(msg.at[j], acc_sp.at[dbuf.at[j]],
                                       ssa, add=True)
                      for j in range(MACRO)]
                for cp in gb:
                    cp.wait()
                sb = [pltpu.async_copy(msg.at[j], acc_sp.at[dbuf.at[j]],
                                       ssb, add=True)
                      for j in range(MACRO, 2 * MACRO)]
                for cp in sa:
                    cp.wait()
                for cp in sb:
                    cp.wait()
                return carry

            lax.fori_loop(0, n_pair, pair, 0)
            plsc.subcore_barrier()
            pltpu.sync_copy(
                acc_sp.at[pl.ds(r0, rows_stage), :],
                s_hbm.at[pl.ds(r0, rows_stage), pl.ds(f0, HHALF)])

        @pl.when(c == 0)
        def _():
            run_half(q0_hbm, 0)

        @pl.when(c == 1)
        def _():
            run_half(q1_hbm, HHALF)

    return edge_kernel


def _make_pool_kernel(n_pad, h):
    rows_tile = n_pad // (NC * NS)
    CH = rows_tile // 4
    FLAT = (G + 1) * h
    CPAD = 80

    @functools.partial(
        pl.kernel,
        out_type=(
            jax.ShapeDtypeStruct((NC * NS, FLAT), jnp.float32),
            jax.ShapeDtypeStruct((NC * NS, FLAT), jnp.float32),
            jax.ShapeDtypeStruct((NC * NS, CPAD), jnp.float32),
        ),
        mesh=_mesh(),
        scratch_types=[
            pltpu.VMEM((CH, h), jnp.float32),
            pltpu.VMEM((rows_tile + LANES,), jnp.int32),
            pltpu.VMEM((FLAT,), jnp.float32),
            pltpu.VMEM((FLAT,), jnp.float32),
            pltpu.VMEM((CPAD,), jnp.float32),
        ],
        compiler_params=pltpu.CompilerParams(use_tc_tiling_on_sc=False,
                                             needs_layout_passes=False),
    )
    def pool_kernel(h_hbm, bi_hbm, maxp, sump, cntp, hbuf, bbuf, mx, sm, ct):
        c = lax.axis_index("c")
        s = lax.axis_index("s")
        t = c * NS + s
        r0 = t * rows_tile
        pltpu.sync_copy(bi_hbm.at[pl.ds(r0, rows_tile)],
                        bbuf.at[pl.ds(0, rows_tile)])

        neg_inf = jnp.full((LANES,), -jnp.inf, jnp.float32)
        zeros = jnp.zeros((LANES,), jnp.float32)

        def ifill(i, carry):
            mx[pl.ds(i * LANES, LANES)] = neg_inf
            sm[pl.ds(i * LANES, LANES)] = zeros
            return carry

        lax.fori_loop(0, FLAT // LANES, ifill, 0)

        def cfill(i, carry):
            ct[pl.ds(i * LANES, LANES)] = zeros
            return carry

        lax.fori_loop(0, CPAD // LANES, cfill, 0)

        iota = lax.iota(jnp.int32, LANES)

        for ci in range(4):
            pltpu.sync_copy(h_hbm.at[pl.ds(r0 + ci * CH, CH), :], hbuf)

            def rowloop(i, carry):
                b = bbuf[pl.ds(ci * CH + i, LANES)][0]
                base = b * h
                for fc in range(h // LANES):
                    idx = base + fc * LANES + iota
                    hrow = hbuf[i, pl.ds(fc * LANES, LANES)]
                    old = plsc.load_gather(mx, [idx])
                    plsc.store_scatter(mx, [idx], jnp.maximum(old, hrow))
                    plsc.addupdate_scatter(sm, [idx], hrow)
                bv = jnp.zeros((LANES,), jnp.int32) + b
                oldc = plsc.load_gather(ct, [bv])
                plsc.store_scatter(ct, [bv], oldc + 1.0)
                return carry

            lax.fori_loop(0, CH, rowloop, 0)

        pltpu.sync_copy(mx, maxp.at[t])
        pltpu.sync_copy(sm, sump.at[t])
        pltpu.sync_copy(ct, cntp.at[t])

    return pool_kernel


def _dinv_from(d):
    deg = d[:, 0:1] + 1.0
    return lax.rsqrt(jnp.maximum(deg, 1.0))


def _q_halves(n_pad):
    return [
        jax.ShapeDtypeStruct((n_pad, HHALF), jnp.float32),
        jax.ShapeDtypeStruct((n_pad, HHALF), jnp.float32),
    ]


def _tc_layer1(n_pad, f_in, h, bs):
    def body(x_ref, w_ref, d_ref, o0_ref, o1_ref):
        dinv = _dinv_from(d_ref[...])
        q = jnp.dot(x_ref[...], w_ref[...],
                    preferred_element_type=jnp.float32) * dinv
        o0_ref[...] = q[:, :HHALF]
        o1_ref[...] = q[:, HHALF:]

    return pl.pallas_call(
        body,
        grid=(n_pad // bs,),
        in_specs=[
            pl.BlockSpec((bs, f_in), lambda i: (i, 0)),
            pl.BlockSpec((f_in, h), lambda i: (0, 0)),
            pl.BlockSpec((bs, LANES), lambda i: (i, 0)),
        ],
        out_specs=[pl.BlockSpec((bs, HHALF), lambda i: (i, 0))] * 2,
        out_shape=_q_halves(n_pad),
    )


def _tc_layer(n_pad, h, bs):
    def body(s_ref, q0_ref, q1_ref, d_ref, b_ref, w_ref, o0_ref, o1_ref):
        dinv = _dinv_from(d_ref[...])
        q = jnp.concatenate([q0_ref[...], q1_ref[...]], axis=1)
        hh = jnp.maximum(dinv * (s_ref[...] + q) + b_ref[...], 0.0)
        qn = jnp.dot(hh, w_ref[...],
                     preferred_element_type=jnp.float32) * dinv
        o0_ref[...] = qn[:, :HHALF]
        o1_ref[...] = qn[:, HHALF:]

    return pl.pallas_call(
        body,
        grid=(n_pad // bs,),
        in_specs=[
            pl.BlockSpec((bs, h), lambda i: (i, 0)),
            pl.BlockSpec((bs, HHALF), lambda i: (i, 0)),
            pl.BlockSpec((bs, HHALF), lambda i: (i, 0)),
            pl.BlockSpec((bs, LANES), lambda i: (i, 0)),
            pl.BlockSpec((1, h), lambda i: (0, 0)),
            pl.BlockSpec((h, h), lambda i: (0, 0)),
        ],
        out_specs=[pl.BlockSpec((bs, HHALF), lambda i: (i, 0))] * 2,
        out_shape=_q_halves(n_pad),
    )


def _tc_final_h(n_pad, h, bs):
    def body(s_ref, q0_ref, q1_ref, d_ref, b_ref, o_ref):
        dinv = _dinv_from(d_ref[...])
        q = jnp.concatenate([q0_ref[...], q1_ref[...]], axis=1)
        o_ref[...] = jnp.maximum(dinv * (s_ref[...] + q) + b_ref[...], 0.0)

    return pl.pallas_call(
        body,
        grid=(n_pad // bs,),
        in_specs=[
            pl.BlockSpec((bs, h), lambda i: (i, 0)),
            pl.BlockSpec((bs, HHALF), lambda i: (i, 0)),
            pl.BlockSpec((bs, HHALF), lambda i: (i, 0)),
            pl.BlockSpec((bs, LANES), lambda i: (i, 0)),
            pl.BlockSpec((1, h), lambda i: (0, 0)),
        ],
        out_specs=pl.BlockSpec((bs, h), lambda i: (i, 0)),
        out_shape=jax.ShapeDtypeStruct((n_pad, h), jnp.float32),
    )


def _tc_pool_reduce(h):
    nt = NC * NS
    CPAD = 80

    def body(maxp_ref, sump_ref, cntp_ref, wo_ref, bo_ref, out_ref, hp_ref):
        hmax = jnp.max(maxp_ref[...], axis=0)[:G, :]
        hsum = jnp.sum(sump_ref[...], axis=0)[:G, :]
        cnt = jnp.sum(cntp_ref[...], axis=0)[:G]
        hmean = hsum / jnp.maximum(cnt, 1.0)[:, None]
        hp = jnp.concatenate([hmax, hmean], axis=1)
        hp_ref[...] = hp
        full = jnp.dot(hp, wo_ref[...], preferred_element_type=jnp.float32)
        out_ref[...] = full[:, 0:1] + bo_ref[...]

    return pl.pallas_call(
        body,
        grid=(1,),
        in_specs=[
            pl.BlockSpec((nt, G + 1, h), lambda i: (0, 0, 0)),
            pl.BlockSpec((nt, G + 1, h), lambda i: (0, 0, 0)),
            pl.BlockSpec((nt, CPAD), lambda i: (0, 0)),
            pl.BlockSpec((2 * h, 128), lambda i: (0, 0)),
            pl.BlockSpec((1, 1), lambda i: (0, 0)),
        ],
        out_specs=[
            pl.BlockSpec((G, 1), lambda i: (0, 0)),
            pl.BlockSpec((G, 2 * h), lambda i: (0, 0)),
        ],
        out_shape=[
            jax.ShapeDtypeStruct((G, 1), jnp.float32),
            jax.ShapeDtypeStruct((G, 2 * h), jnp.float32),
        ],
    )


@jax.jit
def kernel(x, edge_index, batch_index, W1, b1, W2, b2, W3, b3, W4, b4, Wo, bo):
    n, f_in = x.shape
    h = W2.shape[0]
    e = edge_index.shape[1]

    n_pad = -(-n // (8 * NC * NS)) * (8 * NC * NS)
    e_rows = -(-e // (128 * NS * 2 * MACRO)) * (NS * 2 * MACRO)
    e_pad = e_rows * 128

    xp = jnp.pad(x, ((0, n_pad - n), (0, 0)))
    src = jnp.pad(edge_index[0], (0, e_pad - e),
                  constant_values=n).reshape(-1, 128)
    dst = jnp.pad(edge_index[1], (0, e_pad - e),
                  constant_values=n).reshape(-1, 128)
    bip = jnp.pad(batch_index, (0, n_pad - n), constant_values=G)
    zeros = jnp.zeros((n_pad, HHALF), jnp.float32)

    bs = 512
    edge_k = _make_edge_kernel(n_pad, e_rows, h)
    layer_k = _tc_layer(n_pad, h, bs)

    ones_half = jnp.ones((n_pad, HHALF), jnp.float32)
    deg = edge_k(ones_half, ones_half, src, dst, zeros)
    dg = lax.slice(deg, (0, 0), (n_pad, LANES))

    q0, q1 = _tc_layer1(n_pad, f_in, h, bs)(xp, W1, dg)
    s = edge_k(q0, q1, src, dst, zeros)
    for W, b in ((W2, b1), (W3, b2), (W4, b3)):
        q0, q1 = layer_k(s, q0, q1, dg, b.reshape(1, h), W)
        s = edge_k(q0, q1, src, dst, zeros)
    h4 = _tc_final_h(n_pad, h, bs)(s, q0, q1, dg, b4.reshape(1, h))

    maxp, sump, cntp = _make_pool_kernel(n_pad, h)(h4, bip)
    wo_pad = jnp.pad(Wo, ((0, 0), (0, 127)))
    out, hp = _tc_pool_reduce(h)(
        maxp.reshape(NC * NS, G + 1, h), sump.reshape(NC * NS, G + 1, h),
        cntp, wo_pad, bo.reshape(1, 1))
    return (out, hp)


# speedup vs baseline: 1.1866x; 1.1866x over previous
import functools

import jax
import jax.numpy as jnp
from jax import lax
from jax.experimental import pallas as pl
from jax.experimental.pallas import tpu as pltpu
from jax.experimental.pallas import tpu_sc as plsc

NC = 2
NS = 16
LANES = 16
G = 64


def _mesh():
    return plsc.VectorSubcoreMesh(core_axis_name="c", subcore_axis_name="s",
                                  num_cores=NC, num_subcores=NS)


MACRO = 6
HHALF = 32


def _make_edge_kernel(n_pad, e_rows, h):
    rows_stage = n_pad // NS
    erows_tile = e_rows // NS
    n_pair = erows_tile // (2 * MACRO)

    @functools.partial(
        pl.kernel,
        out_type=jax.ShapeDtypeStruct((n_pad, h), jnp.float32),
        mesh=_mesh(),
        scratch_types=[
            pltpu.VMEM_SHARED((n_pad, LANES), jnp.float32),
            pltpu.VMEM_SHARED((n_pad, LANES), jnp.float32),
            pltpu.VMEM((2 * MACRO, 128), jnp.int32),
            pltpu.VMEM((2 * MACRO, 128), jnp.int32),
            pltpu.VMEM((2 * MACRO, 128, LANES), jnp.float32),
            pltpu.SemaphoreType.DMA,
            pltpu.SemaphoreType.DMA,
            pltpu.SemaphoreType.DMA,
            pltpu.SemaphoreType.DMA,
        ],
        compiler_params=pltpu.CompilerParams(use_tc_tiling_on_sc=False),
    )
    def edge_kernel(q0_hbm, q1_hbm, src_hbm, dst_hbm, zeros_hbm, s_hbm,
                    q_sp, acc_sp, sbuf, dbuf, msg, gsa, gsb, ssa, ssb):
        c = lax.axis_index("c")
        s = lax.axis_index("s")
        r0 = s * rows_stage
        e0 = s * erows_tile

        def run_pass(q_hbm, qf0, f0):
            pltpu.sync_copy(
                q_hbm.at[pl.ds(r0, rows_stage), pl.ds(qf0, LANES)],
                q_sp.at[pl.ds(r0, rows_stage), :])
            pltpu.sync_copy(zeros_hbm.at[pl.ds(r0, rows_stage), :],
                            acc_sp.at[pl.ds(r0, rows_stage), :])
            plsc.subcore_barrier()

            def pair(pi, carry):
                row0 = e0 + pi * (2 * MACRO)
                pltpu.sync_copy(src_hbm.at[pl.ds(row0, 2 * MACRO), :], sbuf)
                pltpu.sync_copy(dst_hbm.at[pl.ds(row0, 2 * MACRO), :], dbuf)
                ga = [pltpu.async_copy(q_sp.at[sbuf.at[j]], msg.at[j], gsa)
                      for j in range(MACRO)]
                gb = [pltpu.async_copy(q_sp.at[sbuf.at[j]], msg.at[j], gsb)
                      for j in range(MACRO, 2 * MACRO)]
                for cp in ga:
                    cp.wait()
                sa = [pltpu.async_copy(msg.at[j], acc_sp.at[dbuf.at[j]],
                                       ssa, add=True)
                      for j in range(MACRO)]
                for cp in gb:
                    cp.wait()
                sb = [pltpu.async_copy(msg.at[j], acc_sp.at[dbuf.at[j]],
                                       ssb, add=True)
                      for j in range(MACRO, 2 * MACRO)]
                for cp in sa:
                    cp.wait()
                for cp in sb:
                    cp.wait()
                return carry

            lax.fori_loop(0, n_pair, pair, 0)
            plsc.subcore_barrier()
            pltpu.sync_copy(
                acc_sp.at[pl.ds(r0, rows_stage), :],
                s_hbm.at[pl.ds(r0, rows_stage), pl.ds(f0, LANES)])
            plsc.subcore_barrier()

        @pl.when(c == 0)
        def _():
            run_pass(q0_hbm, 0, 0)
            run_pass(q0_hbm, LANES, LANES)

        @pl.when(c == 1)
        def _():
            run_pass(q1_hbm, 0, 2 * LANES)
            run_pass(q1_hbm, LANES, 3 * LANES)

    return edge_kernel


def _make_pool_kernel(n_pad, h):
    rows_tile = n_pad // (NC * NS)
    CH = rows_tile // 4
    FLAT = (G + 1) * h
    CPAD = 80

    @functools.partial(
        pl.kernel,
        out_type=(
            jax.ShapeDtypeStruct((NC * NS, FLAT), jnp.float32),
            jax.ShapeDtypeStruct((NC * NS, FLAT), jnp.float32),
            jax.ShapeDtypeStruct((NC * NS, CPAD), jnp.float32),
        ),
        mesh=_mesh(),
        scratch_types=[
            pltpu.VMEM((CH, h), jnp.float32),
            pltpu.VMEM((rows_tile + LANES,), jnp.int32),
            pltpu.VMEM((FLAT,), jnp.float32),
            pltpu.VMEM((FLAT,), jnp.float32),
            pltpu.VMEM((CPAD,), jnp.float32),
        ],
        compiler_params=pltpu.CompilerParams(use_tc_tiling_on_sc=False,
                                             needs_layout_passes=False),
    )
    def pool_kernel(h_hbm, bi_hbm, maxp, sump, cntp, hbuf, bbuf, mx, sm, ct):
        c = lax.axis_index("c")
        s = lax.axis_index("s")
        t = c * NS + s
        r0 = t * rows_tile
        pltpu.sync_copy(bi_hbm.at[pl.ds(r0, rows_tile)],
                        bbuf.at[pl.ds(0, rows_tile)])

        neg_inf = jnp.full((LANES,), -jnp.inf, jnp.float32)
        zeros = jnp.zeros((LANES,), jnp.float32)

        def ifill(i, carry):
            mx[pl.ds(i * LANES, LANES)] = neg_inf
            sm[pl.ds(i * LANES, LANES)] = zeros
            return carry

        lax.fori_loop(0, FLAT // LANES, ifill, 0)

        def cfill(i, carry):
            ct[pl.ds(i * LANES, LANES)] = zeros
            return carry

        lax.fori_loop(0, CPAD // LANES, cfill, 0)

        iota = lax.iota(jnp.int32, LANES)

        for ci in range(4):
            pltpu.sync_copy(h_hbm.at[pl.ds(r0 + ci * CH, CH), :], hbuf)

            def rowloop(i, carry):
                b = bbuf[pl.ds(ci * CH + i, LANES)][0]
                base = b * h
                for fc in range(h // LANES):
                    idx = base + fc * LANES + iota
                    hrow = hbuf[i, pl.ds(fc * LANES, LANES)]
                    old = plsc.load_gather(mx, [idx])
                    plsc.store_scatter(mx, [idx], jnp.maximum(old, hrow))
                    plsc.addupdate_scatter(sm, [idx], hrow)
                bv = jnp.zeros((LANES,), jnp.int32) + b
                oldc = plsc.load_gather(ct, [bv])
                plsc.store_scatter(ct, [bv], oldc + 1.0)
                return carry

            lax.fori_loop(0, CH, rowloop, 0)

        pltpu.sync_copy(mx, maxp.at[t])
        pltpu.sync_copy(sm, sump.at[t])
        pltpu.sync_copy(ct, cntp.at[t])

    return pool_kernel


def _dinv_from(d):
    deg = d[:, 0:1] + 1.0
    return lax.rsqrt(jnp.maximum(deg, 1.0))


def _q_halves(n_pad):
    return [
        jax.ShapeDtypeStruct((n_pad, HHALF), jnp.float32),
        jax.ShapeDtypeStruct((n_pad, HHALF), jnp.float32),
    ]


def _tc_layer1(n_pad, f_in, h, bs):
    def body(x_ref, w_ref, d_ref, o0_ref, o1_ref):
        dinv = _dinv_from(d_ref[...])
        q = jnp.dot(x_ref[...], w_ref[...],
                    preferred_element_type=jnp.float32) * dinv
        o0_ref[...] = q[:, :HHALF]
        o1_ref[...] = q[:, HHALF:]

    return pl.pallas_call(
        body,
        grid=(n_pad // bs,),
        in_specs=[
            pl.BlockSpec((bs, f_in), lambda i: (i, 0)),
            pl.BlockSpec((f_in, h), lambda i: (0, 0)),
            pl.BlockSpec((bs, LANES), lambda i: (i, 0)),
        ],
        out_specs=[pl.BlockSpec((bs, HHALF), lambda i: (i, 0))] * 2,
        out_shape=_q_halves(n_pad),
    )


def _tc_layer(n_pad, h, bs):
    def body(s_ref, q0_ref, q1_ref, d_ref, b_ref, w_ref, o0_ref, o1_ref):
        dinv = _dinv_from(d_ref[...])
        q = jnp.concatenate([q0_ref[...], q1_ref[...]], axis=1)
        hh = jnp.maximum(dinv * (s_ref[...] + q) + b_ref[...], 0.0)
        qn = jnp.dot(hh, w_ref[...],
                     preferred_element_type=jnp.float32) * dinv
        o0_ref[...] = qn[:, :HHALF]
        o1_ref[...] = qn[:, HHALF:]

    return pl.pallas_call(
        body,
        grid=(n_pad // bs,),
        in_specs=[
            pl.BlockSpec((bs, h), lambda i: (i, 0)),
            pl.BlockSpec((bs, HHALF), lambda i: (i, 0)),
            pl.BlockSpec((bs, HHALF), lambda i: (i, 0)),
            pl.BlockSpec((bs, LANES), lambda i: (i, 0)),
            pl.BlockSpec((1, h), lambda i: (0, 0)),
            pl.BlockSpec((h, h), lambda i: (0, 0)),
        ],
        out_specs=[pl.BlockSpec((bs, HHALF), lambda i: (i, 0))] * 2,
        out_shape=_q_halves(n_pad),
    )


def _tc_final_h(n_pad, h, bs):
    def body(s_ref, q0_ref, q1_ref, d_ref, b_ref, o_ref):
        dinv = _dinv_from(d_ref[...])
        q = jnp.concatenate([q0_ref[...], q1_ref[...]], axis=1)
        o_ref[...] = jnp.maximum(dinv * (s_ref[...] + q) + b_ref[...], 0.0)

    return pl.pallas_call(
        body,
        grid=(n_pad // bs,),
        in_specs=[
            pl.BlockSpec((bs, h), lambda i: (i, 0)),
            pl.BlockSpec((bs, HHALF), lambda i: (i, 0)),
            pl.BlockSpec((bs, HHALF), lambda i: (i, 0)),
            pl.BlockSpec((bs, LANES), lambda i: (i, 0)),
            pl.BlockSpec((1, h), lambda i: (0, 0)),
        ],
        out_specs=pl.BlockSpec((bs, h), lambda i: (i, 0)),
        out_shape=jax.ShapeDtypeStruct((n_pad, h), jnp.float32),
    )


def _tc_pool_reduce(h):
    nt = NC * NS
    CPAD = 80

    def body(maxp_ref, sump_ref, cntp_ref, wo_ref, bo_ref, out_ref, hp_ref):
        hmax = jnp.max(maxp_ref[...], axis=0)[:G, :]
        hsum = jnp.sum(sump_ref[...], axis=0)[:G, :]
        cnt = jnp.sum(cntp_ref[...], axis=0)[:G]
        hmean = hsum / jnp.maximum(cnt, 1.0)[:, None]
        hp = jnp.concatenate([hmax, hmean], axis=1)
        hp_ref[...] = hp
        full = jnp.dot(hp, wo_ref[...], preferred_element_type=jnp.float32)
        out_ref[...] = full[:, 0:1] + bo_ref[...]

    return pl.pallas_call(
        body,
        grid=(1,),
        in_specs=[
            pl.BlockSpec((nt, G + 1, h), lambda i: (0, 0, 0)),
            pl.BlockSpec((nt, G + 1, h), lambda i: (0, 0, 0)),
            pl.BlockSpec((nt, CPAD), lambda i: (0, 0)),
            pl.BlockSpec((2 * h, 128), lambda i: (0, 0)),
            pl.BlockSpec((1, 1), lambda i: (0, 0)),
        ],
        out_specs=[
            pl.BlockSpec((G, 1), lambda i: (0, 0)),
            pl.BlockSpec((G, 2 * h), lambda i: (0, 0)),
        ],
        out_shape=[
            jax.ShapeDtypeStruct((G, 1), jnp.float32),
            jax.ShapeDtypeStruct((G, 2 * h), jnp.float32),
        ],
    )


@jax.jit
def kernel(x, edge_index, batch_index, W1, b1, W2, b2, W3, b3, W4, b4, Wo, bo):
    n, f_in = x.shape
    h = W2.shape[0]
    e = edge_index.shape[1]

    n_pad = -(-n // (8 * NC * NS)) * (8 * NC * NS)
    e_rows = -(-e // (128 * NS * 2 * MACRO)) * (NS * 2 * MACRO)
    e_pad = e_rows * 128

    xp = jnp.pad(x, ((0, n_pad - n), (0, 0)))
    src = jnp.pad(edge_index[0], (0, e_pad - e),
                  constant_values=n).reshape(-1, 128)
    dst = jnp.pad(edge_index[1], (0, e_pad - e),
                  constant_values=n).reshape(-1, 128)
    bip = jnp.pad(batch_index, (0, n_pad - n), constant_values=G)
    zeros = jnp.zeros((n_pad, LANES), jnp.float32)

    bs = 512
    edge_k = _make_edge_kernel(n_pad, e_rows, h)
    layer_k = _tc_layer(n_pad, h, bs)

    ones_half = jnp.ones((n_pad, HHALF), jnp.float32)
    deg = edge_k(ones_half, ones_half, src, dst, zeros)
    dg = lax.slice(deg, (0, 0), (n_pad, LANES))

    q0, q1 = _tc_layer1(n_pad, f_in, h, bs)(xp, W1, dg)
    s = edge_k(q0, q1, src, dst, zeros)
    for W, b in ((W2, b1), (W3, b2), (W4, b3)):
        q0, q1 = layer_k(s, q0, q1, dg, b.reshape(1, h), W)
        s = edge_k(q0, q1, src, dst, zeros)
    h4 = _tc_final_h(n_pad, h, bs)(s, q0, q1, dg, b4.reshape(1, h))

    maxp, sump, cntp = _make_pool_kernel(n_pad, h)(h4, bip)
    wo_pad = jnp.pad(Wo, ((0, 0), (0, 127)))
    out, hp = _tc_pool_reduce(h)(
        maxp.reshape(NC * NS, G + 1, h), sump.reshape(NC * NS, G + 1, h),
        cntp, wo_pad, bo.reshape(1, 1))
    return (out, hp)


# speedup vs baseline: 1.2942x; 1.0907x over previous
import functools

import jax
import jax.numpy as jnp
from jax import lax
from jax.experimental import pallas as pl
from jax.experimental.pallas import tpu as pltpu
from jax.experimental.pallas import tpu_sc as plsc

NC = 2
NS = 16
LANES = 16
G = 64


def _mesh():
    return plsc.VectorSubcoreMesh(core_axis_name="c", subcore_axis_name="s",
                                  num_cores=NC, num_subcores=NS)


MACRO = 6
HHALF = 32


def _make_edge_kernel(n_pad, e_rows, h):
    rows_stage = n_pad // NS
    erows_tile = e_rows // NS
    n_pair = erows_tile // (2 * MACRO)

    @functools.partial(
        pl.kernel,
        out_type=jax.ShapeDtypeStruct((n_pad, h), jnp.float32),
        mesh=_mesh(),
        scratch_types=[
            pltpu.VMEM_SHARED((n_pad, LANES), jnp.float32),
            pltpu.VMEM_SHARED((n_pad, LANES), jnp.float32),
            pltpu.VMEM((2 * MACRO, 128), jnp.int32),
            pltpu.VMEM((2 * MACRO, 128), jnp.int32),
            pltpu.VMEM((2 * MACRO, 128, LANES), jnp.float32),
            pltpu.SemaphoreType.DMA,
            pltpu.SemaphoreType.DMA,
            pltpu.SemaphoreType.DMA,
            pltpu.SemaphoreType.DMA,
        ],
        compiler_params=pltpu.CompilerParams(use_tc_tiling_on_sc=False),
    )
    def edge_kernel(q0_hbm, q1_hbm, src_hbm, dst_hbm, zeros_hbm, s_hbm,
                    q_sp, acc_sp, sbuf, dbuf, msg, gsa, gsb, ssa, ssb):
        c = lax.axis_index("c")
        s = lax.axis_index("s")
        r0 = s * rows_stage
        e0 = s * erows_tile

        def run_pass(q_hbm, qf0, f0):
            pltpu.sync_copy(
                q_hbm.at[pl.ds(r0, rows_stage), pl.ds(qf0, LANES)],
                q_sp.at[pl.ds(r0, rows_stage), :])
            pltpu.sync_copy(zeros_hbm.at[pl.ds(r0, rows_stage), :],
                            acc_sp.at[pl.ds(r0, rows_stage), :])
            plsc.subcore_barrier()

            def pair(pi, carry):
                row0 = e0 + pi * (2 * MACRO)
                pltpu.sync_copy(src_hbm.at[pl.ds(row0, 2 * MACRO), :], sbuf)
                pltpu.sync_copy(dst_hbm.at[pl.ds(row0, 2 * MACRO), :], dbuf)
                ga = [pltpu.async_copy(q_sp.at[sbuf.at[j]], msg.at[j], gsa)
                      for j in range(MACRO)]
                gb = [pltpu.async_copy(q_sp.at[sbuf.at[j]], msg.at[j], gsb)
                      for j in range(MACRO, 2 * MACRO)]
                for cp in ga:
                    cp.wait()
                sa = [pltpu.async_copy(msg.at[j], acc_sp.at[dbuf.at[j]],
                                       ssa, add=True)
                      for j in range(MACRO)]
                for cp in gb:
                    cp.wait()
                sb = [pltpu.async_copy(msg.at[j], acc_sp.at[dbuf.at[j]],
                                       ssb, add=True)
                      for j in range(MACRO, 2 * MACRO)]
                for cp in sa:
                    cp.wait()
                for cp in sb:
                    cp.wait()
                return carry

            lax.fori_loop(0, n_pair, pair, 0)
            plsc.subcore_barrier()
            pltpu.sync_copy(
                acc_sp.at[pl.ds(r0, rows_stage), :],
                s_hbm.at[pl.ds(r0, rows_stage), pl.ds(f0, LANES)])
            plsc.subcore_barrier()

        @pl.when(c == 0)
        def _():
            run_pass(q0_hbm, 0, 0)
            run_pass(q0_hbm, LANES, LANES)

        @pl.when(c == 1)
        def _():
            run_pass(q1_hbm, 0, 2 * LANES)
            run_pass(q1_hbm, LANES, 3 * LANES)

    return edge_kernel


def _make_deg_kernel(n_pad, e_rows):
    DM = 3
    rows_stage = n_pad // NS
    erows_half = e_rows // NC
    erows_tile = erows_half // NS
    n_pair = erows_tile // (2 * DM)

    @functools.partial(
        pl.kernel,
        out_type=jax.ShapeDtypeStruct((NC, n_pad, LANES), jnp.float32),
        mesh=_mesh(),
        scratch_types=[
            pltpu.VMEM_SHARED((n_pad, LANES), jnp.float32),
            pltpu.VMEM((2 * DM, 128), jnp.int32),
            pltpu.VMEM((128, LANES), jnp.float32),
            pltpu.SemaphoreType.DMA,
            pltpu.SemaphoreType.DMA,
        ],
        compiler_params=pltpu.CompilerParams(use_tc_tiling_on_sc=False),
    )
    def deg_kernel(dst_hbm, zeros_hbm, deg_hbm, acc_sp, dbuf, obuf, ssa, ssb):
        c = lax.axis_index("c")
        s = lax.axis_index("s")
        r0 = s * rows_stage
        e0 = c * erows_half + s * erows_tile

        def ofill(i, carry):
            obuf[i, :] = jnp.ones((LANES,), jnp.float32)
            return carry

        lax.fori_loop(0, 128, ofill, 0)
        pltpu.sync_copy(zeros_hbm.at[pl.ds(r0, rows_stage), :],
                        acc_sp.at[pl.ds(r0, rows_stage), :])
        plsc.subcore_barrier()

        def pair(pi, carry):
            row0 = e0 + pi * (2 * DM)
            pltpu.sync_copy(dst_hbm.at[pl.ds(row0, 2 * DM), :], dbuf)
            sa = [pltpu.async_copy(obuf, acc_sp.at[dbuf.at[j]], ssa, add=True)
                  for j in range(DM)]
            sb = [pltpu.async_copy(obuf, acc_sp.at[dbuf.at[j]], ssb, add=True)
                  for j in range(DM, 2 * DM)]
            for cp in sa:
                cp.wait()
            for cp in sb:
                cp.wait()
            return carry

        lax.fori_loop(0, n_pair, pair, 0)
        plsc.subcore_barrier()
        pltpu.sync_copy(acc_sp.at[pl.ds(r0, rows_stage), :],
                        deg_hbm.at[c, pl.ds(r0, rows_stage), :])

    return deg_kernel


def _make_pool_kernel(n_pad, h):
    rows_tile = n_pad // (NC * NS)
    CH = rows_tile // 4
    FLAT = (G + 1) * h
    CPAD = 80

    @functools.partial(
        pl.kernel,
        out_type=(
            jax.ShapeDtypeStruct((NC * NS, FLAT), jnp.float32),
            jax.ShapeDtypeStruct((NC * NS, FLAT), jnp.float32),
            jax.ShapeDtypeStruct((NC * NS, CPAD), jnp.float32),
        ),
        mesh=_mesh(),
        scratch_types=[
            pltpu.VMEM((CH, h), jnp.float32),
            pltpu.VMEM((rows_tile + LANES,), jnp.int32),
            pltpu.VMEM((FLAT,), jnp.float32),
            pltpu.VMEM((FLAT,), jnp.float32),
            pltpu.VMEM((CPAD,), jnp.float32),
        ],
        compiler_params=pltpu.CompilerParams(use_tc_tiling_on_sc=False,
                                             needs_layout_passes=False),
    )
    def pool_kernel(h_hbm, bi_hbm, maxp, sump, cntp, hbuf, bbuf, mx, sm, ct):
        c = lax.axis_index("c")
        s = lax.axis_index("s")
        t = c * NS + s
        r0 = t * rows_tile
        pltpu.sync_copy(bi_hbm.at[pl.ds(r0, rows_tile)],
                        bbuf.at[pl.ds(0, rows_tile)])

        neg_inf = jnp.full((LANES,), -jnp.inf, jnp.float32)
        zeros = jnp.zeros((LANES,), jnp.float32)

        def ifill(i, carry):
            mx[pl.ds(i * LANES, LANES)] = neg_inf
            sm[pl.ds(i * LANES, LANES)] = zeros
            return carry

        lax.fori_loop(0, FLAT // LANES, ifill, 0)

        def cfill(i, carry):
            ct[pl.ds(i * LANES, LANES)] = zeros
            return carry

        lax.fori_loop(0, CPAD // LANES, cfill, 0)

        iota = lax.iota(jnp.int32, LANES)

        for ci in range(4):
            pltpu.sync_copy(h_hbm.at[pl.ds(r0 + ci * CH, CH), :], hbuf)

            def rowloop(i, carry):
                b = bbuf[pl.ds(ci * CH + i, LANES)][0]
                base = b * h
                for fc in range(h // LANES):
                    idx = base + fc * LANES + iota
                    hrow = hbuf[i, pl.ds(fc * LANES, LANES)]
                    old = plsc.load_gather(mx, [idx])
                    plsc.store_scatter(mx, [idx], jnp.maximum(old, hrow))
                    plsc.addupdate_scatter(sm, [idx], hrow)
                bv = jnp.zeros((LANES,), jnp.int32) + b
                oldc = plsc.load_gather(ct, [bv])
                plsc.store_scatter(ct, [bv], oldc + 1.0)
                return carry

            lax.fori_loop(0, CH, rowloop, 0)

        pltpu.sync_copy(mx, maxp.at[t])
        pltpu.sync_copy(sm, sump.at[t])
        pltpu.sync_copy(ct, cntp.at[t])

    return pool_kernel


def _dinv_from(d0, d1):
    deg = d0[:, 0:1] + d1[:, 0:1] + 1.0
    return lax.rsqrt(jnp.maximum(deg, 1.0))


def _q_halves(n_pad):
    return [
        jax.ShapeDtypeStruct((n_pad, HHALF), jnp.float32),
        jax.ShapeDtypeStruct((n_pad, HHALF), jnp.float32),
    ]


def _tc_layer1(n_pad, f_in, h, bs):
    def body(x_ref, w_ref, d0_ref, d1_ref, o0_ref, o1_ref):
        dinv = _dinv_from(d0_ref[...], d1_ref[...])
        q = jnp.dot(x_ref[...], w_ref[...],
                    preferred_element_type=jnp.float32) * dinv
        o0_ref[...] = q[:, :HHALF]
        o1_ref[...] = q[:, HHALF:]

    return pl.pallas_call(
        body,
        grid=(n_pad // bs,),
        in_specs=[
            pl.BlockSpec((bs, f_in), lambda i: (i, 0)),
            pl.BlockSpec((f_in, h), lambda i: (0, 0)),
            pl.BlockSpec((bs, LANES), lambda i: (i, 0)),
            pl.BlockSpec((bs, LANES), lambda i: (i, 0)),
        ],
        out_specs=[pl.BlockSpec((bs, HHALF), lambda i: (i, 0))] * 2,
        out_shape=_q_halves(n_pad),
    )


def _tc_layer(n_pad, h, bs):
    def body(s_ref, q0_ref, q1_ref, d0_ref, d1_ref, b_ref, w_ref,
             o0_ref, o1_ref):
        dinv = _dinv_from(d0_ref[...], d1_ref[...])
        q = jnp.concatenate([q0_ref[...], q1_ref[...]], axis=1)
        hh = jnp.maximum(dinv * (s_ref[...] + q) + b_ref[...], 0.0)
        qn = jnp.dot(hh, w_ref[...],
                     preferred_element_type=jnp.float32) * dinv
        o0_ref[...] = qn[:, :HHALF]
        o1_ref[...] = qn[:, HHALF:]

    return pl.pallas_call(
        body,
        grid=(n_pad // bs,),
        in_specs=[
            pl.BlockSpec((bs, h), lambda i: (i, 0)),
            pl.BlockSpec((bs, HHALF), lambda i: (i, 0)),
            pl.BlockSpec((bs, HHALF), lambda i: (i, 0)),
            pl.BlockSpec((bs, LANES), lambda i: (i, 0)),
            pl.BlockSpec((bs, LANES), lambda i: (i, 0)),
            pl.BlockSpec((1, h), lambda i: (0, 0)),
            pl.BlockSpec((h, h), lambda i: (0, 0)),
        ],
        out_specs=[pl.BlockSpec((bs, HHALF), lambda i: (i, 0))] * 2,
        out_shape=_q_halves(n_pad),
    )


def _tc_final_h(n_pad, h, bs):
    def body(s_ref, q0_ref, q1_ref, d0_ref, d1_ref, b_ref, o_ref):
        dinv = _dinv_from(d0_ref[...], d1_ref[...])
        q = jnp.concatenate([q0_ref[...], q1_ref[...]], axis=1)
        o_ref[...] = jnp.maximum(dinv * (s_ref[...] + q) + b_ref[...], 0.0)

    return pl.pallas_call(
        body,
        grid=(n_pad // bs,),
        in_specs=[
            pl.BlockSpec((bs, h), lambda i: (i, 0)),
            pl.BlockSpec((bs, HHALF), lambda i: (i, 0)),
            pl.BlockSpec((bs, HHALF), lambda i: (i, 0)),
            pl.BlockSpec((bs, LANES), lambda i: (i, 0)),
            pl.BlockSpec((bs, LANES), lambda i: (i, 0)),
            pl.BlockSpec((1, h), lambda i: (0, 0)),
        ],
        out_specs=pl.BlockSpec((bs, h), lambda i: (i, 0)),
        out_shape=jax.ShapeDtypeStruct((n_pad, h), jnp.float32),
    )


def _tc_pool_reduce(h):
    nt = NC * NS
    CPAD = 80

    def body(maxp_ref, sump_ref, cntp_ref, wo_ref, bo_ref, out_ref, hp_ref):
        hmax = jnp.max(maxp_ref[...], axis=0)[:G, :]
        hsum = jnp.sum(sump_ref[...], axis=0)[:G, :]
        cnt = jnp.sum(cntp_ref[...], axis=0)[:G]
        hmean = hsum / jnp.maximum(cnt, 1.0)[:, None]
        hp = jnp.concatenate([hmax, hmean], axis=1)
        hp_ref[...] = hp
        full = jnp.dot(hp, wo_ref[...], preferred_element_type=jnp.float32)
        out_ref[...] = full[:, 0:1] + bo_ref[...]

    return pl.pallas_call(
        body,
        grid=(1,),
        in_specs=[
            pl.BlockSpec((nt, G + 1, h), lambda i: (0, 0, 0)),
            pl.BlockSpec((nt, G + 1, h), lambda i: (0, 0, 0)),
            pl.BlockSpec((nt, CPAD), lambda i: (0, 0)),
            pl.BlockSpec((2 * h, 128), lambda i: (0, 0)),
            pl.BlockSpec((1, 1), lambda i: (0, 0)),
        ],
        out_specs=[
            pl.BlockSpec((G, 1), lambda i: (0, 0)),
            pl.BlockSpec((G, 2 * h), lambda i: (0, 0)),
        ],
        out_shape=[
            jax.ShapeDtypeStruct((G, 1), jnp.float32),
            jax.ShapeDtypeStruct((G, 2 * h), jnp.float32),
        ],
    )


@jax.jit
def kernel(x, edge_index, batch_index, W1, b1, W2, b2, W3, b3, W4, b4, Wo, bo):
    n, f_in = x.shape
    h = W2.shape[0]
    e = edge_index.shape[1]

    n_pad = -(-n // (8 * NC * NS)) * (8 * NC * NS)
    e_rows = -(-e // (128 * NS * 2 * MACRO)) * (NS * 2 * MACRO)
    e_pad = e_rows * 128

    xp = jnp.pad(x, ((0, n_pad - n), (0, 0)))
    src = jnp.pad(edge_index[0], (0, e_pad - e),
                  constant_values=n).reshape(-1, 128)
    dst = jnp.pad(edge_index[1], (0, e_pad - e),
                  constant_values=n).reshape(-1, 128)
    bip = jnp.pad(batch_index, (0, n_pad - n), constant_values=G)
    zeros = jnp.zeros((n_pad, LANES), jnp.float32)

    bs = 512
    edge_k = _make_edge_kernel(n_pad, e_rows, h)
    layer_k = _tc_layer(n_pad, h, bs)

    deg = _make_deg_kernel(n_pad, e_rows)(dst, zeros)
    d0, d1 = deg[0], deg[1]

    q0, q1 = _tc_layer1(n_pad, f_in, h, bs)(xp, W1, d0, d1)
    s = edge_k(q0, q1, src, dst, zeros)
    for W, b in ((W2, b1), (W3, b2), (W4, b3)):
        q0, q1 = layer_k(s, q0, q1, d0, d1, b.reshape(1, h), W)
        s = edge_k(q0, q1, src, dst, zeros)
    h4 = _tc_final_h(n_pad, h, bs)(s, q0, q1, d0, d1, b4.reshape(1, h))

    maxp, sump, cntp = _make_pool_kernel(n_pad, h)(h4, bip)
    wo_pad = jnp.pad(Wo, ((0, 0), (0, 127)))
    out, hp = _tc_pool_reduce(h)(
        maxp.reshape(NC * NS, G + 1, h), sump.reshape(NC * NS, G + 1, h),
        cntp, wo_pad, bo.reshape(1, 1))
    return (out, hp)
